# Initial kernel scaffold; baseline (speedup 1.0000x reference)
import functools

import jax
import jax.numpy as jnp
from jax import lax
from jax.experimental import pallas as pl
from jax.experimental.pallas import tpu as pltpu
from jax.experimental.pallas import tpu_sc as plsc

N = 10000
E = 320000
D = 128
L = 3
G = 128
NUM_NT = 21
NUM_ET = 4

NC = 2
NS = 16
NW = NC * NS
CHUNK = 128
CH_PER_W = 80
EPW = CHUNK * CH_PER_W
EP = EPW * NW
AGG_ROWS = 10240
ZROWS = AGG_ROWS // NS


def _sc_agg(y, srcp, attrp, dstp):
    mesh = plsc.VectorSubcoreMesh(core_axis_name="c", subcore_axis_name="s")

    @functools.partial(
        pl.kernel,
        out_type=jax.ShapeDtypeStruct((NC, N, D), jnp.float32),
        mesh=mesh,
        scratch_types=[
            pltpu.VMEM((CH_PER_W, CHUNK), jnp.int32),
            pltpu.VMEM((CH_PER_W, CHUNK), jnp.int32),
            pltpu.VMEM((CH_PER_W, CHUNK), jnp.int32),
            pltpu.VMEM((CHUNK, D), jnp.float32),
            pltpu.VMEM_SHARED((AGG_ROWS, D), jnp.float32),
        ],
    )
    def k(y_hbm, src_hbm, attr_hbm, dst_hbm, out_hbm,
          src_v, gidx_v, dst_v, rows_v, agg_sh):
        c = lax.axis_index("c")
        s = lax.axis_index("s")
        wid = c * NS + s

        pltpu.sync_copy(src_hbm.at[wid], src_v)
        pltpu.sync_copy(attr_hbm.at[wid], gidx_v)
        pltpu.sync_copy(dst_hbm.at[wid], dst_v)

        @pl.loop(0, CH_PER_W)
        def _(i):
            for j in range(CHUNK // 16):
                sl = pl.ds(j * 16, 16)
                gidx_v[i, sl] = gidx_v[i, sl] * N + src_v[i, sl]

        @pl.loop(0, CHUNK)
        def _(r):
            for j in range(D // 16):
                rows_v[r, pl.ds(j * 16, 16)] = jnp.zeros((16,), jnp.float32)

        @pl.loop(0, ZROWS // CHUNK)
        def _(zi):
            pltpu.sync_copy(
                rows_v, agg_sh.at[pl.ds(s * ZROWS + zi * CHUNK, CHUNK)])

        plsc.subcore_barrier()

        @pl.loop(0, CH_PER_W)
        def _(i):
            pltpu.sync_copy(y_hbm.at[gidx_v.at[i]], rows_v)
            pltpu.sync_copy(rows_v, agg_sh.at[dst_v.at[i]], add=True)

        plsc.subcore_barrier()

        @pl.when(s < NS - 1)
        def _():
            pltpu.sync_copy(agg_sh.at[pl.ds(s * 640, 640)],
                            out_hbm.at[c, pl.ds(s * 640, 640)])

        @pl.when(s == NS - 1)
        def _():
            pltpu.sync_copy(agg_sh.at[pl.ds(9600, 400)],
                            out_hbm.at[c, pl.ds(9600, 400)])

    return k(y, srcp, attrp, dstp)


def _embed_body(x_ref, w_ref, o_ref):
    oh = (lax.broadcasted_iota(jnp.int32, (N, NUM_NT), 1) == x_ref[...]
          ).astype(jnp.float32)
    o_ref[...] = lax.dot_general(oh, w_ref[...], (((1,), (0,)), ((), ())),
                                 preferred_element_type=jnp.float32)


def _ybuild_body(ew_ref, h_ref, y_ref):
    y_ref[...] = jnp.maximum(h_ref[...] + ew_ref[...], 0.0)


def _mlp_body(h_ref, agg_ref, w1_ref, b1_ref, g1_ref, bb1_ref,
              w2_ref, b2_ref, g2_ref, bb2_ref, eps_ref, o_ref):
    h = h_ref[...]
    z = (1.0 + eps_ref[...]) * h + agg_ref[0] + agg_ref[1]
    u = lax.dot_general(z, w1_ref[...], (((1,), (0,)), ((), ())),
                        preferred_element_type=jnp.float32) + b1_ref[...]
    m = jnp.mean(u, axis=0, keepdims=True)
    v = jnp.mean((u - m) ** 2, axis=0, keepdims=True)
    r = jnp.maximum((u - m) * lax.rsqrt(v + 1e-5) * g1_ref[...] + bb1_ref[...],
                    0.0)
    u2 = lax.dot_general(r, w2_ref[...], (((1,), (0,)), ((), ())),
                         preferred_element_type=jnp.float32) + b2_ref[...]
    m2 = jnp.mean(u2, axis=0, keepdims=True)
    v2 = jnp.mean((u2 - m2) ** 2, axis=0, keepdims=True)
    z2 = jnp.maximum(
        (u2 - m2) * lax.rsqrt(v2 + 1e-5) * g2_ref[...] + bb2_ref[...], 0.0)
    o_ref[...] = z2 + h


def _pool_body(h_ref, b_ref, w1_ref, b1_ref, w2_ref, b2_ref, o_ref):
    oh = (lax.broadcasted_iota(jnp.int32, (G, N), 0) == b_ref[...]
          ).astype(jnp.float32)
    pooled = lax.dot_general(oh, h_ref[...], (((1,), (0,)), ((), ())),
                             preferred_element_type=jnp.float32)
    t = jnp.maximum(
        lax.dot_general(pooled, w1_ref[...], (((1,), (0,)), ((), ())),
                        preferred_element_type=jnp.float32) + b1_ref[...], 0.0)
    o_ref[...] = lax.dot_general(t, w2_ref[...], (((1,), (0,)), ((), ())),
                                 preferred_element_type=jnp.float32) + b2_ref[...]


_NB = 10
_BN = N // _NB


def _ybuild(ew_l, h):
    return pl.pallas_call(
        _ybuild_body,
        grid=(NUM_ET, _NB),
        in_specs=[pl.BlockSpec((1, D), lambda a, i: (a, 0)),
                  pl.BlockSpec((_BN, D), lambda a, i: (i, 0))],
        out_specs=pl.BlockSpec((_BN, D), lambda a, i: (a * _NB + i, 0)),
        out_shape=jax.ShapeDtypeStruct((NUM_ET * N, D), jnp.float32),
    )(ew_l, h)


def kernel(x, edge_index, edge_attr, batch, feat_w, edge_w, lin1_w, lin1_b,
           bn1_g, bn1_b, lin2_w, lin2_b, bn2_g, bn2_b, eps,
           fc1_w, fc1_b, fc2_w, fc2_b):
    src = edge_index[0].astype(jnp.int32)
    dst = edge_index[1].astype(jnp.int32)
    attr = edge_attr.astype(jnp.int32)

    pad = EP - E
    srcp = jnp.concatenate([src, jnp.zeros((pad,), jnp.int32)])
    attrp = jnp.concatenate([attr, jnp.zeros((pad,), jnp.int32)])
    dstp = jnp.concatenate([dst, jnp.full((pad,), N, jnp.int32)])
    srcp = srcp.reshape(NW, CH_PER_W, CHUNK)
    attrp = attrp.reshape(NW, CH_PER_W, CHUNK)
    dstp = dstp.reshape(NW, CH_PER_W, CHUNK)

    h = pl.pallas_call(
        _embed_body,
        out_shape=jax.ShapeDtypeStruct((N, D), jnp.float32),
    )(x.reshape(N, 1).astype(jnp.int32), feat_w)

    for l in range(L):
        yl = _ybuild(edge_w[l], h)
        aggp = _sc_agg(yl, srcp, attrp, dstp)
        h = pl.pallas_call(
            _mlp_body,
            out_shape=jax.ShapeDtypeStruct((N, D), jnp.float32),
        )(h, aggp,
          lin1_w[l], lin1_b[l].reshape(1, D),
          bn1_g[l].reshape(1, D), bn1_b[l].reshape(1, D),
          lin2_w[l], lin2_b[l].reshape(1, D),
          bn2_g[l].reshape(1, D), bn2_b[l].reshape(1, D),
          eps[l].reshape(1, 1))

    out = pl.pallas_call(
        _pool_body,
        out_shape=jax.ShapeDtypeStruct((G, 1), jnp.float32),
    )(h, batch.reshape(1, N).astype(jnp.int32),
      fc1_w, fc1_b.reshape(1, 2 * D), fc2_w, fc2_b.reshape(1, 1))
    return out


# speedup vs baseline: 3.1989x; 3.1989x over previous
import functools

import jax
import jax.numpy as jnp
from jax import lax
from jax.experimental import pallas as pl
from jax.experimental.pallas import tpu as pltpu
from jax.experimental.pallas import tpu_sc as plsc

N = 10000
E = 320000
D = 128
L = 3
G = 128
NUM_NT = 21
NUM_ET = 4

NC = 2
NS = 16
NW = NC * NS
CHUNK = 128
CH_PER_W = 80
EPW = CHUNK * CH_PER_W
EP = EPW * NW
AGG_ROWS = 10240
ZROWS = AGG_ROWS // NS


def _sc_agg(y, srcp, attrp, dstp):
    mesh = plsc.VectorSubcoreMesh(core_axis_name="c", subcore_axis_name="s")

    @functools.partial(
        pl.kernel,
        out_type=jax.ShapeDtypeStruct((NC, N, D), jnp.float32),
        mesh=mesh,
        scratch_types=[
            pltpu.VMEM((CH_PER_W, CHUNK), jnp.int32),
            pltpu.VMEM((CH_PER_W, CHUNK), jnp.int32),
            pltpu.VMEM((CH_PER_W, CHUNK), jnp.int32),
            pltpu.VMEM((CHUNK, D), jnp.float32),
            pltpu.VMEM_SHARED((AGG_ROWS, D), jnp.float32),
        ],
    )
    def k(y_hbm, src_hbm, attr_hbm, dst_hbm, out_hbm,
          src_v, gidx_v, dst_v, rows_v, agg_sh):
        c = lax.axis_index("c")
        s = lax.axis_index("s")
        wid = c * NS + s

        pltpu.sync_copy(src_hbm.at[wid], src_v)
        pltpu.sync_copy(attr_hbm.at[wid], gidx_v)
        pltpu.sync_copy(dst_hbm.at[wid], dst_v)

        @pl.loop(0, CH_PER_W)
        def _(i):
            for j in range(CHUNK // 16):
                sl = pl.ds(j * 16, 16)
                gidx_v[i, sl] = gidx_v[i, sl] * N + src_v[i, sl]

        @pl.loop(0, CHUNK)
        def _(r):
            for j in range(D // 16):
                rows_v[r, pl.ds(j * 16, 16)] = jnp.zeros((16,), jnp.float32)

        @pl.loop(0, ZROWS // CHUNK)
        def _(zi):
            pltpu.sync_copy(
                rows_v, agg_sh.at[pl.ds(s * ZROWS + zi * CHUNK, CHUNK)])

        plsc.subcore_barrier()

        @pl.loop(0, CH_PER_W)
        def _(i):
            pltpu.sync_copy(y_hbm.at[gidx_v.at[i]], rows_v)
            pltpu.sync_copy(rows_v, agg_sh.at[dst_v.at[i]], add=True)

        plsc.subcore_barrier()

        @pl.when(s < NS - 1)
        def _():
            pltpu.sync_copy(agg_sh.at[pl.ds(s * 640, 640)],
                            out_hbm.at[c, pl.ds(s * 640, 640)])

        @pl.when(s == NS - 1)
        def _():
            pltpu.sync_copy(agg_sh.at[pl.ds(9600, 400)],
                            out_hbm.at[c, pl.ds(9600, 400)])

    return k(y, srcp, attrp, dstp)


def _embed_body(x_ref, w_ref, o_ref):
    oh = (lax.broadcasted_iota(jnp.int32, (N, NUM_NT), 1) == x_ref[...]
          ).astype(jnp.float32)
    o_ref[...] = lax.dot_general(oh, w_ref[...], (((1,), (0,)), ((), ())),
                                 preferred_element_type=jnp.float32, precision=lax.Precision.HIGHEST)


def _ybuild_body(ew_ref, h_ref, y_ref):
    y_ref[...] = jnp.maximum(h_ref[...] + ew_ref[0], 0.0)


def _mlp_body(h_ref, agg_ref, w1_ref, b1_ref, g1_ref, bb1_ref,
              w2_ref, b2_ref, g2_ref, bb2_ref, eps_ref, o_ref):
    h = h_ref[...]
    z = (1.0 + eps_ref[...]) * h + agg_ref[0] + agg_ref[1]
    u = lax.dot_general(z, w1_ref[...], (((1,), (0,)), ((), ())),
                        preferred_element_type=jnp.float32) + b1_ref[...]
    m = jnp.mean(u, axis=0, keepdims=True)
    v = jnp.mean((u - m) ** 2, axis=0, keepdims=True)
    r = jnp.maximum((u - m) * lax.rsqrt(v + 1e-5) * g1_ref[...] + bb1_ref[...],
                    0.0)
    u2 = lax.dot_general(r, w2_ref[...], (((1,), (0,)), ((), ())),
                         preferred_element_type=jnp.float32) + b2_ref[...]
    m2 = jnp.mean(u2, axis=0, keepdims=True)
    v2 = jnp.mean((u2 - m2) ** 2, axis=0, keepdims=True)
    z2 = jnp.maximum(
        (u2 - m2) * lax.rsqrt(v2 + 1e-5) * g2_ref[...] + bb2_ref[...], 0.0)
    o_ref[...] = z2 + h


def _pool_body(h_ref, b_ref, w1_ref, b1_ref, w2_ref, b2_ref, o_ref):
    oh = (lax.broadcasted_iota(jnp.int32, (G, N), 0) == b_ref[...]
          ).astype(jnp.float32)
    pooled = lax.dot_general(oh, h_ref[...], (((1,), (0,)), ((), ())),
                             preferred_element_type=jnp.float32, precision=lax.Precision.HIGHEST)
    t = jnp.maximum(
        lax.dot_general(pooled, w1_ref[...], (((1,), (0,)), ((), ())),
                        preferred_element_type=jnp.float32) + b1_ref[...], 0.0)
    o_ref[...] = lax.dot_general(t, w2_ref[...], (((1,), (0,)), ((), ())),
                                 preferred_element_type=jnp.float32) + b2_ref[...]


_NB = 10
_BN = N // _NB


def _ybuild(ew_l, h):
    return pl.pallas_call(
        _ybuild_body,
        grid=(NUM_ET, _NB),
        in_specs=[pl.BlockSpec((1, 1, D), lambda a, i: (a, 0, 0)),
                  pl.BlockSpec((_BN, D), lambda a, i: (i, 0))],
        out_specs=pl.BlockSpec((_BN, D), lambda a, i: (a * _NB + i, 0)),
        out_shape=jax.ShapeDtypeStruct((NUM_ET * N, D), jnp.float32),
    )(ew_l.reshape(NUM_ET, 1, D), h)


def kernel(x, edge_index, edge_attr, batch, feat_w, edge_w, lin1_w, lin1_b,
           bn1_g, bn1_b, lin2_w, lin2_b, bn2_g, bn2_b, eps,
           fc1_w, fc1_b, fc2_w, fc2_b):
    src = edge_index[0].astype(jnp.int32)
    dst = edge_index[1].astype(jnp.int32)
    attr = edge_attr.astype(jnp.int32)

    pad = EP - E
    srcp = jnp.concatenate([src, jnp.zeros((pad,), jnp.int32)])
    attrp = jnp.concatenate([attr, jnp.zeros((pad,), jnp.int32)])
    dstp = jnp.concatenate([dst, jnp.full((pad,), N, jnp.int32)])
    srcp = srcp.reshape(NW, CH_PER_W, CHUNK)
    attrp = attrp.reshape(NW, CH_PER_W, CHUNK)
    dstp = dstp.reshape(NW, CH_PER_W, CHUNK)

    h = pl.pallas_call(
        _embed_body,
        out_shape=jax.ShapeDtypeStruct((N, D), jnp.float32),
    )(x.reshape(N, 1).astype(jnp.int32), feat_w)

    for l in range(L):
        yl = _ybuild(edge_w[l], h)
        aggp = _sc_agg(yl, srcp, attrp, dstp)
        h = pl.pallas_call(
            _mlp_body,
            out_shape=jax.ShapeDtypeStruct((N, D), jnp.float32),
        )(h, aggp,
          lin1_w[l], lin1_b[l].reshape(1, D),
          bn1_g[l].reshape(1, D), bn1_b[l].reshape(1, D),
          lin2_w[l], lin2_b[l].reshape(1, D),
          bn2_g[l].reshape(1, D), bn2_b[l].reshape(1, D),
          eps[l].reshape(1, 1))

    out = pl.pallas_call(
        _pool_body,
        out_shape=jax.ShapeDtypeStruct((G, 1), jnp.float32),
    )(h, batch.reshape(1, N).astype(jnp.int32),
      fc1_w, fc1_b.reshape(1, 2 * D), fc2_w, fc2_b.reshape(1, 1))
    return out


# speedup vs baseline: 8.4274x; 2.6344x over previous
import functools

import jax
import jax.numpy as jnp
from jax import lax
from jax.experimental import pallas as pl
from jax.experimental.pallas import tpu as pltpu
from jax.experimental.pallas import tpu_sc as plsc

N = 10000
E = 320000
D = 128
L = 3
G = 128
NUM_NT = 21
NUM_ET = 4

NC = 2
NS = 16
NW = NC * NS
CHUNK = 128
CH_PER_W = 80
EPW = CHUNK * CH_PER_W
EP = EPW * NW
AGG_ROWS = 10240
ZROWS = AGG_ROWS // NS


def _sc_agg(y, srcp, attrp, dstp):
    mesh = plsc.VectorSubcoreMesh(core_axis_name="c", subcore_axis_name="s")

    @functools.partial(
        pl.kernel,
        out_type=jax.ShapeDtypeStruct((NC, N, D), jnp.float32),
        mesh=mesh,
        scratch_types=[
            pltpu.VMEM((CH_PER_W, CHUNK), jnp.int32),
            pltpu.VMEM((CH_PER_W, CHUNK), jnp.int32),
            pltpu.VMEM((CH_PER_W, CHUNK), jnp.int32),
            pltpu.VMEM((CHUNK, D), jnp.float32),
            pltpu.VMEM_SHARED((AGG_ROWS, D), jnp.float32),
        ],
    )
    def k(y_hbm, src_hbm, attr_hbm, dst_hbm, out_hbm,
          src_v, gidx_v, dst_v, rows_v, agg_sh):
        c = lax.axis_index("c")
        s = lax.axis_index("s")
        wid = c * NS + s

        pltpu.sync_copy(src_hbm.at[wid], src_v)
        pltpu.sync_copy(attr_hbm.at[wid], gidx_v)
        pltpu.sync_copy(dst_hbm.at[wid], dst_v)

        @pl.loop(0, CH_PER_W)
        def _(i):
            for j in range(CHUNK // 16):
                sl = pl.ds(j * 16, 16)
                gidx_v[i, sl] = gidx_v[i, sl] * N + src_v[i, sl]

        @pl.loop(0, CHUNK)
        def _(r):
            for j in range(D // 16):
                rows_v[r, pl.ds(j * 16, 16)] = jnp.zeros((16,), jnp.float32)

        @pl.loop(0, ZROWS // CHUNK)
        def _(zi):
            pltpu.sync_copy(
                rows_v, agg_sh.at[pl.ds(s * ZROWS + zi * CHUNK, CHUNK)])

        plsc.subcore_barrier()

        @pl.loop(0, CH_PER_W)
        def _(i):
            pltpu.sync_copy(y_hbm.at[gidx_v.at[i]], rows_v)
            pltpu.sync_copy(rows_v, agg_sh.at[dst_v.at[i]], add=True)

        plsc.subcore_barrier()

        @pl.when(s < NS - 1)
        def _():
            pltpu.sync_copy(agg_sh.at[pl.ds(s * 640, 640)],
                            out_hbm.at[c, pl.ds(s * 640, 640)])

        @pl.when(s == NS - 1)
        def _():
            pltpu.sync_copy(agg_sh.at[pl.ds(9600, 400)],
                            out_hbm.at[c, pl.ds(9600, 400)])

    return k(y, srcp, attrp, dstp)


def _embed_body(x_ref, w_ref, o_ref):
    oh = (lax.broadcasted_iota(jnp.int32, (N, NUM_NT), 1) == x_ref[...]
          ).astype(jnp.float32)
    o_ref[...] = lax.dot_general(oh, w_ref[...], (((1,), (0,)), ((), ())),
                                 preferred_element_type=jnp.float32, precision=lax.Precision.HIGHEST)


def _ybuild_body(ew_ref, h_ref, y_ref):
    y_ref[...] = jnp.maximum(h_ref[...] + ew_ref[0], 0.0)


def _mlp_body(h_ref, agg_ref, w1_ref, b1_ref, g1_ref, bb1_ref,
              w2_ref, b2_ref, g2_ref, bb2_ref, eps_ref, o_ref):
    h = h_ref[...]
    z = (1.0 + eps_ref[...]) * h + agg_ref[0] + agg_ref[1]
    u = lax.dot_general(z, w1_ref[...], (((1,), (0,)), ((), ())),
                        preferred_element_type=jnp.float32) + b1_ref[...]
    m = jnp.mean(u, axis=0, keepdims=True)
    v = jnp.mean((u - m) ** 2, axis=0, keepdims=True)
    r = jnp.maximum((u - m) * lax.rsqrt(v + 1e-5) * g1_ref[...] + bb1_ref[...],
                    0.0)
    u2 = lax.dot_general(r, w2_ref[...], (((1,), (0,)), ((), ())),
                         preferred_element_type=jnp.float32) + b2_ref[...]
    m2 = jnp.mean(u2, axis=0, keepdims=True)
    v2 = jnp.mean((u2 - m2) ** 2, axis=0, keepdims=True)
    z2 = jnp.maximum(
        (u2 - m2) * lax.rsqrt(v2 + 1e-5) * g2_ref[...] + bb2_ref[...], 0.0)
    o_ref[...] = z2 + h


def _pool_body(h_ref, b_ref, w1_ref, b1_ref, w2_ref, b2_ref, o_ref):
    oh = (lax.broadcasted_iota(jnp.int32, (G, N), 0) == b_ref[...]
          ).astype(jnp.float32)
    pooled = lax.dot_general(oh, h_ref[...], (((1,), (0,)), ((), ())),
                             preferred_element_type=jnp.float32, precision=lax.Precision.HIGHEST)
    t = jnp.maximum(
        lax.dot_general(pooled, w1_ref[...], (((1,), (0,)), ((), ())),
                        preferred_element_type=jnp.float32) + b1_ref[...], 0.0)
    o_ref[...] = lax.dot_general(t, w2_ref[...], (((1,), (0,)), ((), ())),
                                 preferred_element_type=jnp.float32) + b2_ref[...]


_NB = 10
_BN = N // _NB


def _ybuild(ew_l, h):
    return pl.pallas_call(
        _ybuild_body,
        grid=(NUM_ET, _NB),
        in_specs=[pl.BlockSpec((1, 1, D), lambda a, i: (a, 0, 0)),
                  pl.BlockSpec((_BN, D), lambda a, i: (i, 0))],
        out_specs=pl.BlockSpec((_BN, D), lambda a, i: (a * _NB + i, 0)),
        out_shape=jax.ShapeDtypeStruct((NUM_ET * N, D), jnp.float32),
    )(ew_l.reshape(NUM_ET, 1, D), h)


def kernel(x, edge_index, edge_attr, batch, feat_w, edge_w, lin1_w, lin1_b,
           bn1_g, bn1_b, lin2_w, lin2_b, bn2_g, bn2_b, eps,
           fc1_w, fc1_b, fc2_w, fc2_b):
    src = edge_index[0].astype(jnp.int32)
    dst = edge_index[1].astype(jnp.int32)
    attr = edge_attr.astype(jnp.int32)

    pad = EP - E
    pidx = jnp.arange(pad, dtype=jnp.int32)
    srcp = jnp.concatenate([src, pidx % N])
    attrp = jnp.concatenate([attr, pidx % NUM_ET])
    dstp = jnp.concatenate([dst, N + pidx % (AGG_ROWS - N)])
    srcp = srcp.reshape(NW, CH_PER_W, CHUNK)
    attrp = attrp.reshape(NW, CH_PER_W, CHUNK)
    dstp = dstp.reshape(NW, CH_PER_W, CHUNK)

    h = pl.pallas_call(
        _embed_body,
        out_shape=jax.ShapeDtypeStruct((N, D), jnp.float32),
    )(x.reshape(N, 1).astype(jnp.int32), feat_w)

    for l in range(L):
        yl = _ybuild(edge_w[l], h)
        aggp = _sc_agg(yl, srcp, attrp, dstp)
        h = pl.pallas_call(
            _mlp_body,
            out_shape=jax.ShapeDtypeStruct((N, D), jnp.float32),
        )(h, aggp,
          lin1_w[l], lin1_b[l].reshape(1, D),
          bn1_g[l].reshape(1, D), bn1_b[l].reshape(1, D),
          lin2_w[l], lin2_b[l].reshape(1, D),
          bn2_g[l].reshape(1, D), bn2_b[l].reshape(1, D),
          eps[l].reshape(1, 1))

    out = pl.pallas_call(
        _pool_body,
        out_shape=jax.ShapeDtypeStruct((G, 1), jnp.float32),
    )(h, batch.reshape(1, N).astype(jnp.int32),
      fc1_w, fc1_b.reshape(1, 2 * D), fc2_w, fc2_b.reshape(1, 1))
    return out


# speedup vs baseline: 11.5765x; 1.3737x over previous
import functools

import jax
import jax.numpy as jnp
from jax import lax
from jax.experimental import pallas as pl
from jax.experimental.pallas import tpu as pltpu
from jax.experimental.pallas import tpu_sc as plsc

N = 10000
E = 320000
D = 128
L = 3
G = 128
NUM_NT = 21
NUM_ET = 4

NC = 2
NS = 16
NW = NC * NS
CHUNK = 128
CH_PER_W = 80
EPW = CHUNK * CH_PER_W
EP = EPW * NW
AGG_ROWS = 10240
ZROWS = AGG_ROWS // NS


def _sc_agg(y, srcp, attrp, dstp):
    mesh = plsc.VectorSubcoreMesh(core_axis_name="c", subcore_axis_name="s")

    @functools.partial(
        pl.kernel,
        out_type=jax.ShapeDtypeStruct((NC, N, D), jnp.float32),
        mesh=mesh,
        scratch_types=[
            pltpu.VMEM((CH_PER_W // 2, CHUNK), jnp.int32),
            pltpu.VMEM((CH_PER_W // 2, CHUNK), jnp.int32),
            pltpu.VMEM((CHUNK, D), jnp.float32),
            pltpu.VMEM((CHUNK, D), jnp.float32),
            pltpu.VMEM_SHARED((AGG_ROWS, D), jnp.float32),
            pltpu.SemaphoreType.DMA,
            pltpu.SemaphoreType.DMA,
        ],
    )
    def k(y_hbm, src_hbm, attr_hbm, dst_hbm, out_hbm,
          dst_v, gidx_v, rows_v, rows1_v, agg_sh, gs0, gs1):
        c = lax.axis_index("c")
        s = lax.axis_index("s")
        wid = c * NS + s
        half = CH_PER_W // 2

        @pl.loop(0, CHUNK)
        def _(r):
            for j in range(D // 16):
                rows_v[r, pl.ds(j * 16, 16)] = jnp.zeros((16,), jnp.float32)

        @pl.loop(0, ZROWS // CHUNK)
        def _(zi):
            pltpu.sync_copy(
                rows_v, agg_sh.at[pl.ds(s * ZROWS + zi * CHUNK, CHUNK)])

        plsc.subcore_barrier()

        for h in range(2):
            hs = pl.ds(h * half, half)
            pltpu.sync_copy(src_hbm.at[wid, hs], dst_v)
            pltpu.sync_copy(attr_hbm.at[wid, hs], gidx_v)

            @pl.loop(0, half)
            def _(i):
                for j in range(CHUNK // 16):
                    sl = pl.ds(j * 16, 16)
                    gidx_v[i, sl] = gidx_v[i, sl] * N + dst_v[i, sl]

            pltpu.sync_copy(dst_hbm.at[wid, hs], dst_v)

            pltpu.async_copy(y_hbm.at[gidx_v.at[0]], rows_v, gs0)

            @pl.loop(0, half // 2)
            def _(p):
                a = 2 * p
                b = a + 1
                pltpu.async_copy(y_hbm.at[gidx_v.at[b]], rows1_v, gs1)
                pltpu.make_async_copy(y_hbm.at[gidx_v.at[a]], rows_v, gs0).wait()
                pltpu.sync_copy(rows_v, agg_sh.at[dst_v.at[a]], add=True)

                @pl.when(a + 2 < half)
                def _():
                    pltpu.async_copy(y_hbm.at[gidx_v.at[a + 2]], rows_v, gs0)

                pltpu.make_async_copy(y_hbm.at[gidx_v.at[b]], rows1_v, gs1).wait()
                pltpu.sync_copy(rows1_v, agg_sh.at[dst_v.at[b]], add=True)

        plsc.subcore_barrier()

        @pl.when(s < NS - 1)
        def _():
            pltpu.sync_copy(agg_sh.at[pl.ds(s * 640, 640)],
                            out_hbm.at[c, pl.ds(s * 640, 640)])

        @pl.when(s == NS - 1)
        def _():
            pltpu.sync_copy(agg_sh.at[pl.ds(9600, 400)],
                            out_hbm.at[c, pl.ds(9600, 400)])

    return k(y, srcp, attrp, dstp)


def _embed_body(x_ref, w_ref, o_ref):
    oh = (lax.broadcasted_iota(jnp.int32, (N, NUM_NT), 1) == x_ref[...]
          ).astype(jnp.float32)
    o_ref[...] = lax.dot_general(oh, w_ref[...], (((1,), (0,)), ((), ())),
                                 preferred_element_type=jnp.float32, precision=lax.Precision.HIGHEST)


def _ybuild_body(ew_ref, h_ref, y_ref):
    y_ref[...] = jnp.maximum(h_ref[...] + ew_ref[0], 0.0)


def _mlp_body(h_ref, agg_ref, w1_ref, b1_ref, g1_ref, bb1_ref,
              w2_ref, b2_ref, g2_ref, bb2_ref, eps_ref, o_ref):
    h = h_ref[...]
    z = (1.0 + eps_ref[...]) * h + agg_ref[0] + agg_ref[1]
    u = lax.dot_general(z, w1_ref[...], (((1,), (0,)), ((), ())),
                        preferred_element_type=jnp.float32) + b1_ref[...]
    m = jnp.mean(u, axis=0, keepdims=True)
    v = jnp.mean((u - m) ** 2, axis=0, keepdims=True)
    r = jnp.maximum((u - m) * lax.rsqrt(v + 1e-5) * g1_ref[...] + bb1_ref[...],
                    0.0)
    u2 = lax.dot_general(r, w2_ref[...], (((1,), (0,)), ((), ())),
                         preferred_element_type=jnp.float32) + b2_ref[...]
    m2 = jnp.mean(u2, axis=0, keepdims=True)
    v2 = jnp.mean((u2 - m2) ** 2, axis=0, keepdims=True)
    z2 = jnp.maximum(
        (u2 - m2) * lax.rsqrt(v2 + 1e-5) * g2_ref[...] + bb2_ref[...], 0.0)
    o_ref[...] = z2 + h


def _pool_body(h_ref, b_ref, w1_ref, b1_ref, w2_ref, b2_ref, o_ref):
    oh = (lax.broadcasted_iota(jnp.int32, (G, N), 0) == b_ref[...]
          ).astype(jnp.float32)
    pooled = lax.dot_general(oh, h_ref[...], (((1,), (0,)), ((), ())),
                             preferred_element_type=jnp.float32, precision=lax.Precision.HIGHEST)
    t = jnp.maximum(
        lax.dot_general(pooled, w1_ref[...], (((1,), (0,)), ((), ())),
                        preferred_element_type=jnp.float32) + b1_ref[...], 0.0)
    o_ref[...] = lax.dot_general(t, w2_ref[...], (((1,), (0,)), ((), ())),
                                 preferred_element_type=jnp.float32) + b2_ref[...]


_NB = 10
_BN = N // _NB


def _ybuild(ew_l, h):
    return pl.pallas_call(
        _ybuild_body,
        grid=(NUM_ET, _NB),
        in_specs=[pl.BlockSpec((1, 1, D), lambda a, i: (a, 0, 0)),
                  pl.BlockSpec((_BN, D), lambda a, i: (i, 0))],
        out_specs=pl.BlockSpec((_BN, D), lambda a, i: (a * _NB + i, 0)),
        out_shape=jax.ShapeDtypeStruct((NUM_ET * N, D), jnp.float32),
    )(ew_l.reshape(NUM_ET, 1, D), h)


def kernel(x, edge_index, edge_attr, batch, feat_w, edge_w, lin1_w, lin1_b,
           bn1_g, bn1_b, lin2_w, lin2_b, bn2_g, bn2_b, eps,
           fc1_w, fc1_b, fc2_w, fc2_b):
    src = edge_index[0].astype(jnp.int32)
    dst = edge_index[1].astype(jnp.int32)
    attr = edge_attr.astype(jnp.int32)

    pad = EP - E
    pidx = jnp.arange(pad, dtype=jnp.int32)
    srcp = jnp.concatenate([src, pidx % N])
    attrp = jnp.concatenate([attr, pidx % NUM_ET])
    dstp = jnp.concatenate([dst, N + pidx % (AGG_ROWS - N)])
    srcp = srcp.reshape(NW, CH_PER_W, CHUNK)
    attrp = attrp.reshape(NW, CH_PER_W, CHUNK)
    dstp = dstp.reshape(NW, CH_PER_W, CHUNK)

    h = pl.pallas_call(
        _embed_body,
        out_shape=jax.ShapeDtypeStruct((N, D), jnp.float32),
    )(x.reshape(N, 1).astype(jnp.int32), feat_w)

    for l in range(L):
        yl = _ybuild(edge_w[l], h)
        aggp = _sc_agg(yl, srcp, attrp, dstp)
        h = pl.pallas_call(
            _mlp_body,
            out_shape=jax.ShapeDtypeStruct((N, D), jnp.float32),
        )(h, aggp,
          lin1_w[l], lin1_b[l].reshape(1, D),
          bn1_g[l].reshape(1, D), bn1_b[l].reshape(1, D),
          lin2_w[l], lin2_b[l].reshape(1, D),
          bn2_g[l].reshape(1, D), bn2_b[l].reshape(1, D),
          eps[l].reshape(1, 1))

    out = pl.pallas_call(
        _pool_body,
        out_shape=jax.ShapeDtypeStruct((G, 1), jnp.float32),
    )(h, batch.reshape(1, N).astype(jnp.int32),
      fc1_w, fc1_b.reshape(1, 2 * D), fc2_w, fc2_b.reshape(1, 1))
    return out


# speedup vs baseline: 13.4474x; 1.1616x over previous
import functools

import jax
import jax.numpy as jnp
from jax import lax
from jax.experimental import pallas as pl
from jax.experimental.pallas import tpu as pltpu
from jax.experimental.pallas import tpu_sc as plsc

N = 10000
E = 320000
D = 128
L = 3
G = 128
NUM_NT = 21
NUM_ET = 4

NC = 2
NS = 16
NW = NC * NS
CHUNK = 128
CH_PER_W = 80
EPW = CHUNK * CH_PER_W
EP = EPW * NW
AGG_ROWS = 10240
ZROWS = AGG_ROWS // NS


def _sc_agg(y, srcp, attrp, dstp):
    mesh = plsc.VectorSubcoreMesh(core_axis_name="c", subcore_axis_name="s")

    @functools.partial(
        pl.kernel,
        out_type=jax.ShapeDtypeStruct((NC, N, D), jnp.float32),
        mesh=mesh,
        scratch_types=[
            pltpu.VMEM((CH_PER_W // 2, CHUNK), jnp.int32),
            pltpu.VMEM((CH_PER_W // 2, CHUNK), jnp.int32),
            pltpu.VMEM((CHUNK, D), jnp.float32),
            pltpu.VMEM((CHUNK, D), jnp.float32),
            pltpu.VMEM_SHARED((AGG_ROWS, D), jnp.float32),
            pltpu.SemaphoreType.DMA,
            pltpu.SemaphoreType.DMA,
        ],
    )
    def k(y_hbm, src_hbm, attr_hbm, dst_hbm, out_hbm,
          dst_v, gidx_v, rows_v, rows1_v, agg_sh, gs0, gs1):
        c = lax.axis_index("c")
        s = lax.axis_index("s")
        wid = c * NS + s
        half = CH_PER_W // 2

        @pl.loop(0, CHUNK)
        def _(r):
            for j in range(D // 16):
                rows_v[r, pl.ds(j * 16, 16)] = jnp.zeros((16,), jnp.float32)

        @pl.loop(0, ZROWS // CHUNK)
        def _(zi):
            pltpu.sync_copy(
                rows_v, agg_sh.at[pl.ds(s * ZROWS + zi * CHUNK, CHUNK)])

        plsc.subcore_barrier()

        for h in range(2):
            hs = pl.ds(h * half, half)
            pltpu.sync_copy(src_hbm.at[wid, hs], dst_v)
            pltpu.sync_copy(attr_hbm.at[wid, hs], gidx_v)

            @pl.loop(0, half)
            def _(i):
                for j in range(CHUNK // 16):
                    sl = pl.ds(j * 16, 16)
                    gidx_v[i, sl] = gidx_v[i, sl] * N + dst_v[i, sl]

            pltpu.sync_copy(dst_hbm.at[wid, hs], dst_v)

            pltpu.async_copy(y_hbm.at[gidx_v.at[0]], rows_v, gs0)

            @pl.loop(0, half // 2)
            def _(p):
                a = 2 * p
                b = a + 1
                pltpu.async_copy(y_hbm.at[gidx_v.at[b]], rows1_v, gs1)
                pltpu.make_async_copy(y_hbm.at[gidx_v.at[a]], rows_v, gs0).wait()
                pltpu.sync_copy(rows_v, agg_sh.at[dst_v.at[a]], add=True)

                @pl.when(a + 2 < half)
                def _():
                    pltpu.async_copy(y_hbm.at[gidx_v.at[a + 2]], rows_v, gs0)

                pltpu.make_async_copy(y_hbm.at[gidx_v.at[b]], rows1_v, gs1).wait()
                pltpu.sync_copy(rows1_v, agg_sh.at[dst_v.at[b]], add=True)

        plsc.subcore_barrier()

        @pl.when(s < NS - 1)
        def _():
            pltpu.sync_copy(agg_sh.at[pl.ds(s * 640, 640)],
                            out_hbm.at[c, pl.ds(s * 640, 640)])

        @pl.when(s == NS - 1)
        def _():
            pltpu.sync_copy(agg_sh.at[pl.ds(9600, 400)],
                            out_hbm.at[c, pl.ds(9600, 400)])

    return k(y, srcp, attrp, dstp)


def _write_y(y_ref, h, ew):
    for a in range(NUM_ET):
        y_ref[a] = jnp.maximum(h + ew[a, :][None, :], 0.0)


def _embed_y_body(x_ref, w_ref, ew_ref, h_ref, y_ref):
    oh = (lax.broadcasted_iota(jnp.int32, (N, NUM_NT), 1) == x_ref[...]
          ).astype(jnp.float32)
    h = lax.dot_general(oh, w_ref[...], (((1,), (0,)), ((), ())),
                        preferred_element_type=jnp.float32,
                        precision=lax.Precision.HIGHEST)
    h_ref[...] = h
    _write_y(y_ref, h, ew_ref[...])


def _mlp_math(h, agg_ref, w1_ref, b1_ref, g1_ref, bb1_ref,
              w2_ref, b2_ref, g2_ref, bb2_ref, eps_ref):
    z = (1.0 + eps_ref[...]) * h + agg_ref[0] + agg_ref[1]
    u = lax.dot_general(z, w1_ref[...], (((1,), (0,)), ((), ())),
                        preferred_element_type=jnp.float32) + b1_ref[...]
    m = jnp.mean(u, axis=0, keepdims=True)
    v = jnp.mean((u - m) ** 2, axis=0, keepdims=True)
    r = jnp.maximum((u - m) * lax.rsqrt(v + 1e-5) * g1_ref[...] + bb1_ref[...],
                    0.0)
    u2 = lax.dot_general(r, w2_ref[...], (((1,), (0,)), ((), ())),
                         preferred_element_type=jnp.float32) + b2_ref[...]
    m2 = jnp.mean(u2, axis=0, keepdims=True)
    v2 = jnp.mean((u2 - m2) ** 2, axis=0, keepdims=True)
    z2 = jnp.maximum(
        (u2 - m2) * lax.rsqrt(v2 + 1e-5) * g2_ref[...] + bb2_ref[...], 0.0)
    return z2 + h


def _mlp_y_body(h_ref, agg_ref, w1_ref, b1_ref, g1_ref, bb1_ref,
                w2_ref, b2_ref, g2_ref, bb2_ref, eps_ref, ew_ref,
                o_ref, y_ref):
    h_next = _mlp_math(h_ref[...], agg_ref, w1_ref, b1_ref, g1_ref, bb1_ref,
                       w2_ref, b2_ref, g2_ref, bb2_ref, eps_ref)
    o_ref[...] = h_next
    _write_y(y_ref, h_next, ew_ref[...])


def _mlp_pool_body(h_ref, agg_ref, w1_ref, b1_ref, g1_ref, bb1_ref,
                   w2_ref, b2_ref, g2_ref, bb2_ref, eps_ref,
                   b_ref, f1_ref, fb1_ref, f2_ref, fb2_ref, o_ref):
    h_next = _mlp_math(h_ref[...], agg_ref, w1_ref, b1_ref, g1_ref, bb1_ref,
                       w2_ref, b2_ref, g2_ref, bb2_ref, eps_ref)
    oh = (lax.broadcasted_iota(jnp.int32, (G, N), 0) == b_ref[...]
          ).astype(jnp.float32)
    pooled = lax.dot_general(oh, h_next, (((1,), (0,)), ((), ())),
                             preferred_element_type=jnp.float32,
                             precision=lax.Precision.HIGHEST)
    t = jnp.maximum(
        lax.dot_general(pooled, f1_ref[...], (((1,), (0,)), ((), ())),
                        preferred_element_type=jnp.float32) + fb1_ref[...], 0.0)
    o_ref[...] = lax.dot_general(t, f2_ref[...], (((1,), (0,)), ((), ())),
                                 preferred_element_type=jnp.float32) + fb2_ref[...]


def kernel(x, edge_index, edge_attr, batch, feat_w, edge_w, lin1_w, lin1_b,
           bn1_g, bn1_b, lin2_w, lin2_b, bn2_g, bn2_b, eps,
           fc1_w, fc1_b, fc2_w, fc2_b):
    src = edge_index[0].astype(jnp.int32)
    dst = edge_index[1].astype(jnp.int32)
    attr = edge_attr.astype(jnp.int32)

    pad = EP - E
    pidx = jnp.arange(pad, dtype=jnp.int32)
    srcp = jnp.concatenate([src, pidx % N])
    attrp = jnp.concatenate([attr, pidx % NUM_ET])
    dstp = jnp.concatenate([dst, N + pidx % (AGG_ROWS - N)])
    srcp = srcp.reshape(NW, CH_PER_W, CHUNK)
    attrp = attrp.reshape(NW, CH_PER_W, CHUNK)
    dstp = dstp.reshape(NW, CH_PER_W, CHUNK)

    h, y = pl.pallas_call(
        _embed_y_body,
        out_shape=[jax.ShapeDtypeStruct((N, D), jnp.float32),
                   jax.ShapeDtypeStruct((NUM_ET, N, D), jnp.float32)],
    )(x.reshape(N, 1).astype(jnp.int32), feat_w, edge_w[0])

    for l in range(L - 1):
        aggp = _sc_agg(y.reshape(NUM_ET * N, D), srcp, attrp, dstp)
        h, y = pl.pallas_call(
            _mlp_y_body,
            out_shape=[jax.ShapeDtypeStruct((N, D), jnp.float32),
                       jax.ShapeDtypeStruct((NUM_ET, N, D), jnp.float32)],
        )(h, aggp,
          lin1_w[l], lin1_b[l].reshape(1, D),
          bn1_g[l].reshape(1, D), bn1_b[l].reshape(1, D),
          lin2_w[l], lin2_b[l].reshape(1, D),
          bn2_g[l].reshape(1, D), bn2_b[l].reshape(1, D),
          eps[l].reshape(1, 1), edge_w[l + 1])

    aggp = _sc_agg(y.reshape(NUM_ET * N, D), srcp, attrp, dstp)
    out = pl.pallas_call(
        _mlp_pool_body,
        out_shape=jax.ShapeDtypeStruct((G, 1), jnp.float32),
    )(h, aggp,
      lin1_w[2], lin1_b[2].reshape(1, D),
      bn1_g[2].reshape(1, D), bn1_b[2].reshape(1, D),
      lin2_w[2], lin2_b[2].reshape(1, D),
      bn2_g[2].reshape(1, D), bn2_b[2].reshape(1, D),
      eps[2].reshape(1, 1),
      batch.reshape(1, N).astype(jnp.int32),
      fc1_w, fc1_b.reshape(1, 2 * D), fc2_w, fc2_b.reshape(1, 1))
    return out
